# Initial kernel scaffold; baseline (speedup 1.0000x reference)
import jax
import jax.numpy as jnp
from jax.experimental import pallas as pl


def kernel(x, keep_ratio, ln_w, ln_b, w, b):
    raise NotImplementedError("write your pallas kernel here")


# speedup vs baseline: 9.0139x; 9.0139x over previous
import functools

import jax
import jax.numpy as jnp
from jax import lax
from jax.experimental import pallas as pl
from jax.experimental.pallas import tpu as pltpu
from jax.experimental.pallas import tpu_sc as plsc

_B, _N, _C = 4, 8192, 1024
_K = int(_N * 0.7)
_CH = 48
_NMAIN = _K // _CH
_TAIL_START = _K - _CH
_NCHUNK = _NMAIN + 1
_KIDX_W = 5808
_TAIL_OFF = 5760
_NW = 32
_CPW = (_B * _NCHUNK) // _NW


def _scores_body(x_ref, lnw_ref, lnb_ref, w_ref, b_ref, out_ref):
    xb = x_ref[0]
    mu = jnp.mean(xb, axis=-1, keepdims=True)
    xc = xb - mu
    var = jnp.mean(xc * xc, axis=-1, keepdims=True)
    xn = xc / jnp.sqrt(var + 1e-5) * lnw_ref[0] + lnb_ref[0]
    xnr = xn.astype(jnp.bfloat16).astype(jnp.float32)
    wr = w_ref[0].astype(jnp.float32)
    out_ref[0, 0, 0, :] = jnp.sum(xnr * wr, axis=-1) + b_ref[0, 0]


def _scores(x, ln_w, ln_b, w, b):
    TN = 512
    grid = (_B, _N // TN)
    return pl.pallas_call(
        _scores_body,
        grid=grid,
        in_specs=[
            pl.BlockSpec((1, TN, _C), lambda i, j: (i, j, 0)),
            pl.BlockSpec((1, _C), lambda i, j: (0, 0)),
            pl.BlockSpec((1, _C), lambda i, j: (0, 0)),
            pl.BlockSpec((1, _C), lambda i, j: (0, 0)),
            pl.BlockSpec((1, 1), lambda i, j: (0, 0)),
        ],
        out_specs=pl.BlockSpec((1, 1, 1, TN), lambda i, j: (i, j, 0, 0)),
        out_shape=jax.ShapeDtypeStruct((_B, _N // TN, 1, TN), jnp.float32),
        compiler_params=pltpu.CompilerParams(
            dimension_semantics=("parallel", "parallel")),
    )(x, ln_w.reshape(1, _C), ln_b.reshape(1, _C),
      w.reshape(1, _C).astype(jnp.bfloat16),
      b.reshape(1, 1)).reshape(_B, _N)


def _excl_cumsum(m):
    m3 = m.reshape(_B, _N // 128, 128)
    r = lax.broadcasted_iota(jnp.int32, (128, 128), 0)
    c = lax.broadcasted_iota(jnp.int32, (128, 128), 1)
    U = (r <= c).astype(jnp.float32)
    within = lax.dot_general(
        m3.reshape(-1, 128), U, (((1,), (0,)), ((), ())),
        preferred_element_type=jnp.float32).reshape(_B, _N // 128, 128)
    rowsum = within[:, :, 127]
    nb = _N // 128
    rb = lax.broadcasted_iota(jnp.int32, (nb, nb), 0)
    cb = lax.broadcasted_iota(jnp.int32, (nb, nb), 1)
    S = (rb < cb).astype(jnp.float32)
    roff = lax.dot_general(rowsum, S, (((1,), (0,)), ((), ())),
                           preferred_element_type=jnp.float32)
    return (within + roff[:, :, None] - m3).reshape(_B, _N)


def _select_body(s_ref, out_ref):
    s = s_ref[...]
    imin = jnp.full((), -2147483648, jnp.int32)
    u = lax.bitcast_convert_type(s, jnp.int32)
    key = jnp.where(u >= 0, u, imin - u)

    def step(_, carry):
        lo, hi = carry
        active = lo < hi
        mid = (lo & hi) + ((lo ^ hi) >> 1)
        cnt = jnp.sum((key > mid).astype(jnp.int32), axis=1, keepdims=True)
        cond = cnt >= _K
        lo = jnp.where(active & cond, mid + 1, lo)
        hi = jnp.where(active & (~cond), mid, hi)
        return lo, hi

    lo0 = jnp.full((_B, 1), -2147483648, jnp.int32)
    hi0 = jnp.full((_B, 1), 2147483647, jnp.int32)
    T, _ = lax.fori_loop(0, 32, step, (lo0, hi0))

    gt = key > T
    eq = key == T
    cnt_gt = jnp.sum(gt.astype(jnp.int32), axis=1, keepdims=True)
    need_eq = _K - cnt_gt
    eq_rank = _excl_cumsum(eq.astype(jnp.float32)).astype(jnp.int32)
    keep = gt | (eq & (eq_rank < need_eq))
    pos = _excl_cumsum(keep.astype(jnp.float32)).astype(jnp.int32)
    out_ref[...] = jnp.where(keep, pos, jnp.int32(-1))


def _select(scores):
    return pl.pallas_call(
        _select_body,
        out_shape=jax.ShapeDtypeStruct((_B, _N), jnp.int32),
    )(scores)


def _compact_body(pose_hbm, kidx_hbm, pose_v, kidx_v):
    wid = lax.axis_index("s") * 2 + lax.axis_index("c")

    @pl.when(wid < _B)
    def _():
        b = wid
        pltpu.sync_copy(pose_hbm.at[pl.ds(b * _N, _N)], pose_v)

        def body(j, carry):
            base = j * 16
            pos = pose_v[pl.ds(base, 16)]
            g = b * _N + base + lax.iota(jnp.int32, 16)
            posc = jnp.maximum(pos, 0)
            plsc.store_scatter(kidx_v, [posc], g, mask=pos >= 0)
            tail = pos >= _TAIL_START
            toff = posc - _TAIL_START + _TAIL_OFF
            plsc.store_scatter(kidx_v, [jnp.where(tail, toff, 0)], g, mask=tail)
            return carry

        lax.fori_loop(0, _N // 16, body, 0)
        pltpu.sync_copy(kidx_v, kidx_hbm.at[pl.ds(b * _KIDX_W, _KIDX_W)])


def _compact(pos_enc):
    mesh = plsc.VectorSubcoreMesh(core_axis_name="c", subcore_axis_name="s")
    f = pl.kernel(
        _compact_body,
        out_type=jax.ShapeDtypeStruct((_B * _KIDX_W,), jnp.int32),
        mesh=mesh,
        scratch_types=[
            pltpu.VMEM((_N,), jnp.int32),
            pltpu.VMEM((_KIDX_W,), jnp.int32),
        ],
        compiler_params=pltpu.CompilerParams(needs_layout_passes=False),
    )
    return f(pos_enc.reshape(_B * _N))


def _gather_body(x_hbm, kidx_hbm, out_hbm, idx_v, opos_v, rows_v, sem):
    wid = lax.axis_index("s") * 2 + lax.axis_index("c")
    for t in range(_CPW):
        c = wid * _CPW + t
        b = c // _NCHUNK
        i = c - b * _NCHUNK
        is_main = i < _NMAIN
        src = b * _KIDX_W + jnp.where(is_main, i * _CH, _TAIL_OFF)
        dst = b * _K + jnp.where(is_main, i * _CH, _TAIL_START)
        for jj in range(_CH // 16):
            opos_v[pl.ds(jj * 16, 16)] = dst + jj * 16 + lax.iota(jnp.int32, 16)
        pltpu.sync_copy(kidx_hbm.at[pl.ds(src, _CH)], idx_v)
        pltpu.async_copy(x_hbm.at[idx_v], rows_v, sem).wait()
        pltpu.async_copy(rows_v, out_hbm.at[opos_v], sem).wait()


def _gather(x2d, kidx):
    mesh = plsc.VectorSubcoreMesh(core_axis_name="c", subcore_axis_name="s")
    f = pl.kernel(
        _gather_body,
        out_type=jax.ShapeDtypeStruct((_B * _K, _C), jnp.float32),
        mesh=mesh,
        scratch_types=[
            pltpu.VMEM((_CH,), jnp.int32),
            pltpu.VMEM((_CH,), jnp.int32),
            pltpu.VMEM((_CH, _C), jnp.float32),
            pltpu.SemaphoreType.DMA,
        ],
        compiler_params=pltpu.CompilerParams(needs_layout_passes=False),
    )
    return f(x2d, kidx)


def kernel(x, keep_ratio, ln_w, ln_b, w, b):
    B, N, C = x.shape
    assert (B, N, C) == (_B, _N, _C)
    scores = _scores(x, ln_w, ln_b, w[0], b)
    pos_enc = _select(scores)
    kidx = _compact(pos_enc)
    out2d = _gather(x.reshape(B * N, C), kidx)
    return out2d.reshape(B, _K, C)


# speedup vs baseline: 9.3818x; 1.0408x over previous
import functools

import jax
import jax.numpy as jnp
from jax import lax
from jax.experimental import pallas as pl
from jax.experimental.pallas import tpu as pltpu
from jax.experimental.pallas import tpu_sc as plsc

_B, _N, _C = 4, 8192, 1024
_K = int(_N * 0.7)
_CH = 48
_NMAIN = _K // _CH
_TAIL_START = _K - _CH
_NCHUNK = _NMAIN + 1
_KIDX_W = 5808
_TAIL_OFF = 5760
_NW = 32
_CPW = (_B * _NCHUNK) // _NW


def _scores_body(x_ref, lnw_ref, lnb_ref, w_ref, b_ref, out_ref):
    xb = x_ref[0]
    mu = jnp.mean(xb, axis=-1, keepdims=True)
    xc = xb - mu
    var = jnp.mean(xc * xc, axis=-1, keepdims=True)
    xn = xc / jnp.sqrt(var + 1e-5) * lnw_ref[0] + lnb_ref[0]
    xnr = xn.astype(jnp.bfloat16).astype(jnp.float32)
    wr = w_ref[0].astype(jnp.float32)
    out_ref[0, 0, 0, :] = jnp.sum(xnr * wr, axis=-1) + b_ref[0, 0]


def _scores(x, ln_w, ln_b, w, b):
    TN = 512
    grid = (_B, _N // TN)
    return pl.pallas_call(
        _scores_body,
        grid=grid,
        in_specs=[
            pl.BlockSpec((1, TN, _C), lambda i, j: (i, j, 0)),
            pl.BlockSpec((1, _C), lambda i, j: (0, 0)),
            pl.BlockSpec((1, _C), lambda i, j: (0, 0)),
            pl.BlockSpec((1, _C), lambda i, j: (0, 0)),
            pl.BlockSpec((1, 1), lambda i, j: (0, 0)),
        ],
        out_specs=pl.BlockSpec((1, 1, 1, TN), lambda i, j: (i, j, 0, 0)),
        out_shape=jax.ShapeDtypeStruct((_B, _N // TN, 1, TN), jnp.float32),
        compiler_params=pltpu.CompilerParams(
            dimension_semantics=("parallel", "parallel")),
    )(x, ln_w.reshape(1, _C), ln_b.reshape(1, _C),
      w.reshape(1, _C).astype(jnp.bfloat16),
      b.reshape(1, 1)).reshape(_B, _N)


def _excl_cumsum(m):
    m3 = m.reshape(_B, _N // 128, 128)
    r = lax.broadcasted_iota(jnp.int32, (128, 128), 0)
    c = lax.broadcasted_iota(jnp.int32, (128, 128), 1)
    U = (r <= c).astype(jnp.float32)
    within = lax.dot_general(
        m3.reshape(-1, 128), U, (((1,), (0,)), ((), ())),
        preferred_element_type=jnp.float32).reshape(_B, _N // 128, 128)
    rowsum = within[:, :, 127]
    nb = _N // 128
    rb = lax.broadcasted_iota(jnp.int32, (nb, nb), 0)
    cb = lax.broadcasted_iota(jnp.int32, (nb, nb), 1)
    S = (rb < cb).astype(jnp.float32)
    roff = lax.dot_general(rowsum, S, (((1,), (0,)), ((), ())),
                           preferred_element_type=jnp.float32)
    return (within + roff[:, :, None] - m3).reshape(_B, _N)


def _select_body(s_ref, out_ref):
    s = s_ref[...]
    imin = jnp.full((), -2147483648, jnp.int32)
    u = lax.bitcast_convert_type(s, jnp.int32)
    key = jnp.where(u >= 0, u, imin - u)

    def step(_, carry):
        lo, hi = carry
        active = lo < hi
        mid = (lo & hi) + ((lo ^ hi) >> 1)
        cnt = jnp.sum((key > mid).astype(jnp.int32), axis=1, keepdims=True)
        cond = cnt >= _K
        lo = jnp.where(active & cond, mid + 1, lo)
        hi = jnp.where(active & (~cond), mid, hi)
        return lo, hi

    lo0 = jnp.full((_B, 1), -2147483648, jnp.int32)
    hi0 = jnp.full((_B, 1), 2147483647, jnp.int32)
    T, _ = lax.fori_loop(0, 32, step, (lo0, hi0))

    gt = key > T
    eq = key == T
    cnt_gt = jnp.sum(gt.astype(jnp.int32), axis=1, keepdims=True)
    need_eq = _K - cnt_gt
    eq_rank = _excl_cumsum(eq.astype(jnp.float32)).astype(jnp.int32)
    keep = gt | (eq & (eq_rank < need_eq))
    pos = _excl_cumsum(keep.astype(jnp.float32)).astype(jnp.int32)
    out_ref[...] = jnp.where(keep, pos, jnp.int32(-1))


def _select(scores):
    return pl.pallas_call(
        _select_body,
        out_shape=jax.ShapeDtypeStruct((_B, _N), jnp.int32),
    )(scores)


def _compact_body(pose_hbm, kidx_hbm, pose_v, kidx_v):
    wid = lax.axis_index("s") * 2 + lax.axis_index("c")

    @pl.when(wid < _B)
    def _():
        b = wid
        pltpu.sync_copy(pose_hbm.at[pl.ds(b * _N, _N)], pose_v)

        def body(j, carry):
            base = j * 16
            pos = pose_v[pl.ds(base, 16)]
            g = base + lax.iota(jnp.int32, 16)
            posc = jnp.maximum(pos, 0)
            plsc.store_scatter(kidx_v, [posc], g, mask=pos >= 0)
            tail = pos >= _TAIL_START
            toff = posc - _TAIL_START + _TAIL_OFF
            plsc.store_scatter(kidx_v, [jnp.where(tail, toff, 0)], g, mask=tail)
            return carry

        lax.fori_loop(0, _N // 16, body, 0)
        pltpu.sync_copy(kidx_v, kidx_hbm.at[pl.ds(b * _KIDX_W, _KIDX_W)])


def _compact(pos_enc):
    mesh = plsc.VectorSubcoreMesh(core_axis_name="c", subcore_axis_name="s")
    f = pl.kernel(
        _compact_body,
        out_type=jax.ShapeDtypeStruct((_B * _KIDX_W,), jnp.int32),
        mesh=mesh,
        scratch_types=[
            pltpu.VMEM((_N,), jnp.int32),
            pltpu.VMEM((_KIDX_W,), jnp.int32),
        ],
        compiler_params=pltpu.CompilerParams(needs_layout_passes=False),
    )
    return f(pos_enc.reshape(_B * _N))


def _gather_body(x_hbm, kidx_hbm, out_hbm, idxall_v, opos0, opos1,
                 rows0, rows1, isem, gs0, gs1, ss0, ss1):
    wid = lax.axis_index("s") * 2 + lax.axis_index("c")

    def chunk_info(t):
        c = wid * _CPW + t
        b = c // _NCHUNK
        i = c - b * _NCHUNK
        is_main = i < _NMAIN
        src = b * _KIDX_W + jnp.where(is_main, i * _CH, _TAIL_OFF)
        dst = b * _K + jnp.where(is_main, i * _CH, _TAIL_START)
        return b, src, dst

    handles = []
    binfo = []
    for t in range(_CPW):
        b, src, dst = chunk_info(t)
        binfo.append((b, dst))
        handles.append(pltpu.async_copy(
            kidx_hbm.at[pl.ds(src, _CH)],
            idxall_v.at[pl.ds(t * _CH, _CH)], isem))
    for h in handles:
        h.wait()

    rows = [rows0, rows1]
    opos = [opos0, opos1]
    gsem = [gs0, gs1]
    ssem = [ss0, ss1]
    gh = [None, None]
    sh = [None, None]

    def fill_opos(buf, dst):
        for jj in range(_CH // 16):
            buf[pl.ds(jj * 16, 16)] = dst + jj * 16 + lax.iota(jnp.int32, 16)

    for t in range(_CPW):
        bi = t % 2
        b, dst = binfo[t]
        if sh[bi] is not None:
            sh[bi].wait()
        gh[bi] = pltpu.async_copy(
            x_hbm.at[b].at[idxall_v.at[pl.ds(t * _CH, _CH)]], rows[bi], gsem[bi])
        if t >= 1:
            pi = 1 - bi
            pb, pdst = binfo[t - 1]
            fill_opos(opos[pi], pdst)
            gh[pi].wait()
            sh[pi] = pltpu.async_copy(rows[pi], out_hbm.at[opos[pi]], ssem[pi])
    li = (_CPW - 1) % 2
    lb, ldst = binfo[_CPW - 1]
    fill_opos(opos[li], ldst)
    gh[li].wait()
    sh[li] = pltpu.async_copy(rows[li], out_hbm.at[opos[li]], ssem[li])
    sh[li].wait()
    sh[1 - li].wait()


def _gather(x, kidx):
    mesh = plsc.VectorSubcoreMesh(core_axis_name="c", subcore_axis_name="s")
    f = pl.kernel(
        _gather_body,
        out_type=jax.ShapeDtypeStruct((_B * _K, _C), jnp.float32),
        mesh=mesh,
        scratch_types=[
            pltpu.VMEM((_CPW * _CH,), jnp.int32),
            pltpu.VMEM((_CH,), jnp.int32),
            pltpu.VMEM((_CH,), jnp.int32),
            pltpu.VMEM((_CH, _C), jnp.float32),
            pltpu.VMEM((_CH, _C), jnp.float32),
            pltpu.SemaphoreType.DMA,
            pltpu.SemaphoreType.DMA,
            pltpu.SemaphoreType.DMA,
            pltpu.SemaphoreType.DMA,
            pltpu.SemaphoreType.DMA,
        ],
        compiler_params=pltpu.CompilerParams(needs_layout_passes=False),
    )
    return f(x, kidx)


def kernel(x, keep_ratio, ln_w, ln_b, w, b):
    B, N, C = x.shape
    assert (B, N, C) == (_B, _N, _C)
    scores = _scores(x, ln_w, ln_b, w[0], b)
    pos_enc = _select(scores)
    kidx = _compact(pos_enc)
    out2d = _gather(x, kidx)
    return out2d.reshape(B, _K, C)


# speedup vs baseline: 10.5459x; 1.1241x over previous
import functools

import jax
import jax.numpy as jnp
from jax import lax
from jax.experimental import pallas as pl
from jax.experimental.pallas import tpu as pltpu
from jax.experimental.pallas import tpu_sc as plsc

_B, _N, _C = 4, 8192, 1024
_K = int(_N * 0.7)
_CH = 48
_NMAIN = _K // _CH
_TAIL_START = _K - _CH
_NCHUNK = _NMAIN + 1
_KIDX_W = 5808
_TAIL_OFF = 5760
_NW = 32
_CPW = (_B * _NCHUNK) // _NW


def _scores_body(x_ref, lnw_ref, lnb_ref, w_ref, b_ref, out_ref):
    xb = x_ref[0]
    mu = jnp.mean(xb, axis=-1, keepdims=True)
    xc = xb - mu
    var = jnp.mean(xc * xc, axis=-1, keepdims=True)
    xn = xc / jnp.sqrt(var + 1e-5) * lnw_ref[0] + lnb_ref[0]
    xnr = xn.astype(jnp.bfloat16).astype(jnp.float32)
    wr = w_ref[0].astype(jnp.float32)
    out_ref[0, 0, 0, :] = jnp.sum(xnr * wr, axis=-1) + b_ref[0, 0]


def _scores(x, ln_w, ln_b, w, b):
    TN = 512
    grid = (_B, _N // TN)
    return pl.pallas_call(
        _scores_body,
        grid=grid,
        in_specs=[
            pl.BlockSpec((1, TN, _C), lambda i, j: (i, j, 0)),
            pl.BlockSpec((1, _C), lambda i, j: (0, 0)),
            pl.BlockSpec((1, _C), lambda i, j: (0, 0)),
            pl.BlockSpec((1, _C), lambda i, j: (0, 0)),
            pl.BlockSpec((1, 1), lambda i, j: (0, 0)),
        ],
        out_specs=pl.BlockSpec((1, 1, 1, TN), lambda i, j: (i, j, 0, 0)),
        out_shape=jax.ShapeDtypeStruct((_B, _N // TN, 1, TN), jnp.float32),
        compiler_params=pltpu.CompilerParams(
            dimension_semantics=("parallel", "parallel")),
    )(x, ln_w.reshape(1, _C), ln_b.reshape(1, _C),
      w.reshape(1, _C).astype(jnp.bfloat16),
      b.reshape(1, 1)).reshape(_B, _N)


def _excl_cumsum(m):
    m3 = m.reshape(_B, _N // 128, 128)
    r = lax.broadcasted_iota(jnp.int32, (128, 128), 0)
    c = lax.broadcasted_iota(jnp.int32, (128, 128), 1)
    U = (r <= c).astype(jnp.float32)
    within = lax.dot_general(
        m3.reshape(-1, 128), U, (((1,), (0,)), ((), ())),
        preferred_element_type=jnp.float32).reshape(_B, _N // 128, 128)
    rowsum = within[:, :, 127]
    nb = _N // 128
    rb = lax.broadcasted_iota(jnp.int32, (nb, nb), 0)
    cb = lax.broadcasted_iota(jnp.int32, (nb, nb), 1)
    S = (rb < cb).astype(jnp.float32)
    roff = lax.dot_general(rowsum, S, (((1,), (0,)), ((), ())),
                           preferred_element_type=jnp.float32)
    return (within + roff[:, :, None] - m3).reshape(_B, _N)


def _select_body(s_ref, out_ref):
    s = s_ref[...]
    imin = jnp.full((), -2147483648, jnp.int32)
    u = lax.bitcast_convert_type(s, jnp.int32)
    key = jnp.where(u >= 0, u, imin - u)

    def step(_, carry):
        lo, hi = carry
        active = lo < hi
        mid = (lo & hi) + ((lo ^ hi) >> 1)
        cnt = jnp.sum((key > mid).astype(jnp.int32), axis=1, keepdims=True)
        cond = cnt >= _K
        lo = jnp.where(active & cond, mid + 1, lo)
        hi = jnp.where(active & (~cond), mid, hi)
        return lo, hi

    lo0 = jnp.full((_B, 1), -2147483648, jnp.int32)
    hi0 = jnp.full((_B, 1), 2147483647, jnp.int32)
    T, _ = lax.fori_loop(0, 32, step, (lo0, hi0))

    gt = key > T
    eq = key == T
    cnt_gt = jnp.sum(gt.astype(jnp.int32), axis=1, keepdims=True)
    need_eq = _K - cnt_gt
    eq_rank = _excl_cumsum(eq.astype(jnp.float32)).astype(jnp.int32)
    keep = gt | (eq & (eq_rank < need_eq))
    pos = _excl_cumsum(keep.astype(jnp.float32)).astype(jnp.int32)
    out_ref[...] = jnp.where(keep, pos, jnp.int32(-1))


def _select(scores):
    return pl.pallas_call(
        _select_body,
        out_shape=jax.ShapeDtypeStruct((_B, _N), jnp.int32),
    )(scores)


def _compact_body(pose_hbm, kidx_hbm, pose_v, kidx_v):
    wid = lax.axis_index("s") * 2 + lax.axis_index("c")

    @pl.when(wid < _B)
    def _():
        b = wid
        pltpu.sync_copy(pose_hbm.at[pl.ds(b * _N, _N)], pose_v)

        def body(j, carry):
            base = j * 16
            pos = pose_v[pl.ds(base, 16)]
            g = base + lax.iota(jnp.int32, 16)
            posc = jnp.maximum(pos, 0)
            plsc.store_scatter(kidx_v, [posc], g, mask=pos >= 0)
            tail = pos >= _TAIL_START
            toff = posc - _TAIL_START + _TAIL_OFF
            plsc.store_scatter(kidx_v, [jnp.where(tail, toff, 0)], g, mask=tail)
            return carry

        lax.fori_loop(0, _N // 16, body, 0)
        pltpu.sync_copy(kidx_v, kidx_hbm.at[pl.ds(b * _KIDX_W, _KIDX_W)])


def _compact(pos_enc):
    mesh = plsc.VectorSubcoreMesh(core_axis_name="c", subcore_axis_name="s")
    f = pl.kernel(
        _compact_body,
        out_type=jax.ShapeDtypeStruct((_B * _KIDX_W,), jnp.int32),
        mesh=mesh,
        scratch_types=[
            pltpu.VMEM((_N,), jnp.int32),
            pltpu.VMEM((_KIDX_W,), jnp.int32),
        ],
        compiler_params=pltpu.CompilerParams(needs_layout_passes=False),
    )
    return f(pos_enc.reshape(_B * _N))


def _gather_body(x_hbm, kidx_hbm, out_hbm, idxall_v, opos0, opos1,
                 rows0, rows1, isem, gs0, gs1, ss0, ss1):
    wid = lax.axis_index("s") * 2 + lax.axis_index("c")

    def chunk_info(t):
        c = wid * _CPW + t
        b = c // _NCHUNK
        i = c - b * _NCHUNK
        is_main = i < _NMAIN
        src = b * _KIDX_W + jnp.where(is_main, i * _CH, _TAIL_OFF)
        dst = jnp.where(is_main, i * _CH, _TAIL_START)
        return b, src, dst

    handles = []
    binfo = []
    for t in range(_CPW):
        b, src, dst = chunk_info(t)
        binfo.append((b, dst))
        handles.append(pltpu.async_copy(
            kidx_hbm.at[pl.ds(src, _CH)],
            idxall_v.at[pl.ds(t * _CH, _CH)], isem))
    for h in handles:
        h.wait()

    rows = [rows0, rows1]
    opos = [opos0, opos1]
    gsem = [gs0, gs1]
    ssem = [ss0, ss1]
    gh = [None, None]
    sh = [None, None]

    def fill_opos(buf, dst):
        for jj in range(_CH // 16):
            buf[pl.ds(jj * 16, 16)] = dst + jj * 16 + lax.iota(jnp.int32, 16)

    for t in range(_CPW):
        bi = t % 2
        b, dst = binfo[t]
        if sh[bi] is not None:
            sh[bi].wait()
        gh[bi] = pltpu.async_copy(
            x_hbm.at[b].at[idxall_v.at[pl.ds(t * _CH, _CH)]], rows[bi], gsem[bi])
        if t >= 1:
            pi = 1 - bi
            pb, pdst = binfo[t - 1]
            fill_opos(opos[pi], pdst)
            gh[pi].wait()
            sh[pi] = pltpu.async_copy(rows[pi], out_hbm.at[pb].at[opos[pi]], ssem[pi])
    li = (_CPW - 1) % 2
    lb, ldst = binfo[_CPW - 1]
    fill_opos(opos[li], ldst)
    gh[li].wait()
    sh[li] = pltpu.async_copy(rows[li], out_hbm.at[lb].at[opos[li]], ssem[li])
    sh[li].wait()
    sh[1 - li].wait()


def _gather(x, kidx):
    mesh = plsc.VectorSubcoreMesh(core_axis_name="c", subcore_axis_name="s")
    f = pl.kernel(
        _gather_body,
        out_type=jax.ShapeDtypeStruct((_B, _K, _C), jnp.float32),
        mesh=mesh,
        scratch_types=[
            pltpu.VMEM((_CPW * _CH,), jnp.int32),
            pltpu.VMEM((_CH,), jnp.int32),
            pltpu.VMEM((_CH,), jnp.int32),
            pltpu.VMEM((_CH, _C), jnp.float32),
            pltpu.VMEM((_CH, _C), jnp.float32),
            pltpu.SemaphoreType.DMA,
            pltpu.SemaphoreType.DMA,
            pltpu.SemaphoreType.DMA,
            pltpu.SemaphoreType.DMA,
            pltpu.SemaphoreType.DMA,
        ],
        compiler_params=pltpu.CompilerParams(needs_layout_passes=False),
    )
    return f(x, kidx)


def kernel(x, keep_ratio, ln_w, ln_b, w, b):
    B, N, C = x.shape
    assert (B, N, C) == (_B, _N, _C)
    scores = _scores(x, ln_w, ln_b, w[0], b)
    pos_enc = _select(scores)
    kidx = _compact(pos_enc)
    return _gather(x, kidx)
